# Initial kernel scaffold; baseline (speedup 1.0000x reference)
import jax
import jax.numpy as jnp
from jax.experimental import pallas as pl


def kernel(x, real_weights, imag_weights):
    raise NotImplementedError("write your pallas kernel here")


# speedup vs baseline: 362.2783x; 362.2783x over previous
import jax
import jax.numpy as jnp
from jax.experimental import pallas as pl

_C = 64
_OUT = 64
_ROW_TILE = 2048


def _body(x_ref, wr_ref, wi_ref, o_ref):
    w_eff = 2.0 * (wr_ref[0:64, :] - wi_ref[64:128, :] - wr_ref[128:192, :]
                   + wi_ref[192:256, :] + wr_ref[256:320, :])
    xb = x_ref[0]
    o_ref[0] = jax.lax.dot_general(
        xb, w_eff, (((0,), (0,)), ((), ())),
        preferred_element_type=jnp.float32)


def kernel(x, real_weights, imag_weights):
    N, C, m, _ = x.shape
    M = m * m
    xr = x.reshape(N, C, M)
    r = _ROW_TILE if M % _ROW_TILE == 0 else M
    out = pl.pallas_call(
        _body,
        grid=(N, M // r),
        in_specs=[
            pl.BlockSpec((1, C, r), lambda n, j: (n, 0, j)),
            pl.BlockSpec(real_weights.shape, lambda n, j: (0, 0)),
            pl.BlockSpec(imag_weights.shape, lambda n, j: (0, 0)),
        ],
        out_specs=pl.BlockSpec((1, r, _OUT), lambda n, j: (n, j, 0)),
        out_shape=jax.ShapeDtypeStruct((N, M, _OUT), jnp.float32),
    )(xr, real_weights, imag_weights)
    return out.reshape(N, m, m, _OUT)


# speedup vs baseline: 362.8979x; 1.0017x over previous
import jax
import jax.numpy as jnp
from jax.experimental import pallas as pl
from jax.experimental.pallas import tpu as pltpu

_C = 64
_OUT = 64
_ROW_TILE = 2048


def _body(x_ref, wr_ref, wi_ref, o_ref):
    w_eff = 2.0 * (wr_ref[0:64, :] - wi_ref[64:128, :] - wr_ref[128:192, :]
                   + wi_ref[192:256, :] + wr_ref[256:320, :])
    xb = x_ref[0]
    o_ref[0] = jax.lax.dot_general(
        xb, w_eff, (((0,), (0,)), ((), ())),
        preferred_element_type=jnp.float32)


def kernel(x, real_weights, imag_weights):
    N, C, m, _ = x.shape
    M = m * m
    xr = x.reshape(N, C, M)
    r = _ROW_TILE if M % _ROW_TILE == 0 else M
    out = pl.pallas_call(
        _body,
        grid=(N, M // r),
        in_specs=[
            pl.BlockSpec((1, C, r), lambda n, j: (n, 0, j)),
            pl.BlockSpec(real_weights.shape, lambda n, j: (0, 0)),
            pl.BlockSpec(imag_weights.shape, lambda n, j: (0, 0)),
        ],
        out_specs=pl.BlockSpec((1, r, _OUT), lambda n, j: (n, j, 0)),
        out_shape=jax.ShapeDtypeStruct((N, M, _OUT), jnp.float32),
        compiler_params=pltpu.CompilerParams(
            dimension_semantics=("parallel", "parallel")),
    )(xr, real_weights, imag_weights)
    return out.reshape(N, m, m, _OUT)
